# Initial kernel scaffold; baseline (speedup 1.0000x reference)
import jax
import jax.numpy as jnp
from jax.experimental import pallas as pl


def kernel(x, edge_index, batch, W_in, b_in, W_h0, b_h0, W_h1, b_h1, fc1_W, fc1_b, fc2_W, fc2_b, fc3_W, fc3_b):
    raise NotImplementedError("write your pallas kernel here")


# speedup vs baseline: 5.9660x; 5.9660x over previous
import functools
import jax
import jax.numpy as jnp
from jax import lax
from jax.experimental import pallas as pl
from jax.experimental.pallas import tpu as pltpu
from jax.experimental.pallas import tpu_sc as plsc

N = 10000
E = 320000
D = 128
G = 64

NC = 2
NS = 16
NH = 5120
NHT = NH + 8
SR = NH // NS
ET = E // NS
CH = 80
NCH = ET // CH


def _remap_dst(didx_v, c):
    base = c * NH
    for k in range(CH // 16):
        v = didx_v[pl.ds(k * 16, 16)] - base
        ok = (v >= 0) & (v < NH)
        didx_v[pl.ds(k * 16, 16)] = jnp.where(ok, v, NH)


def _deg_body(dst_hbm, out_hbm, acc_sh, didx_v, ones_v, zb_v):
    c = lax.axis_index("c")
    s = lax.axis_index("s")
    base = s * ET

    one16 = jnp.full((16,), 1.0, jnp.float32)
    zero16 = jnp.zeros((16,), jnp.float32)

    def fill(i, _):
        ones_v[i] = one16
        return 0
    lax.fori_loop(0, CH, fill, 0)

    def zfill(i, _):
        zb_v[i] = zero16
        return 0
    lax.fori_loop(0, SR, zfill, 0)
    pltpu.sync_copy(zb_v, acc_sh.at[pl.ds(s * SR, SR)])

    @pl.when(s == NS - 1)
    def _():
        pltpu.sync_copy(zb_v.at[pl.ds(0, 8)], acc_sh.at[pl.ds(NH, 8)])

    plsc.subcore_barrier()

    def chunk(j, _):
        pltpu.sync_copy(dst_hbm.at[pl.ds(base + j * CH, CH)], didx_v)
        _remap_dst(didx_v, c)
        pltpu.sync_copy(ones_v, acc_sh.at[didx_v], add=True)
        return 0
    lax.fori_loop(0, NCH, chunk, 0)
    plsc.subcore_barrier()

    pltpu.sync_copy(acc_sh.at[pl.ds(s * SR, SR)],
                    out_hbm.at[pl.ds(c * NH + s * SR, SR)])


@functools.cache
def _sc_mesh():
    return plsc.VectorSubcoreMesh(core_axis_name="c", subcore_axis_name="s",
                                  num_cores=NC, num_subcores=NS)


@functools.cache
def _deg_kernel():
    return pl.kernel(
        _deg_body,
        out_type=jax.ShapeDtypeStruct((2 * NH, 16), jnp.float32),
        mesh=_sc_mesh(),
        scratch_types=[
            pltpu.VMEM_SHARED((NHT, 16), jnp.float32),
            pltpu.VMEM((CH,), jnp.int32),
            pltpu.VMEM((CH, 16), jnp.float32),
            pltpu.VMEM((SR, 16), jnp.float32),
        ],
    )


def _edge_body(z_hbm, src_hbm, dst_hbm, out_hbm,
               acc_sh, sidx_v, didx_v, rows_v, zb_v, sem):
    c = lax.axis_index("c")
    s = lax.axis_index("s")
    base = s * ET

    zero16 = jnp.zeros((16,), jnp.float32)

    def zfill(i, _):
        for j in range(8):
            zb_v[i, pl.ds(j * 16, 16)] = zero16
        return 0
    lax.fori_loop(0, SR, zfill, 0)
    pltpu.sync_copy(zb_v, acc_sh.at[pl.ds(s * SR, SR)])

    @pl.when(s == NS - 1)
    def _():
        pltpu.sync_copy(zb_v.at[pl.ds(0, 8)], acc_sh.at[pl.ds(NH, 8)])

    plsc.subcore_barrier()

    def chunk(j, _):
        off = base + j * CH
        pltpu.sync_copy(src_hbm.at[pl.ds(off, CH)], sidx_v)
        pltpu.sync_copy(dst_hbm.at[pl.ds(off, CH)], didx_v)
        _remap_dst(didx_v, c)
        pltpu.async_copy(z_hbm.at[sidx_v], rows_v, sem).wait()
        pltpu.sync_copy(rows_v, acc_sh.at[didx_v], add=True)
        return 0
    lax.fori_loop(0, NCH, chunk, 0)
    plsc.subcore_barrier()

    pltpu.sync_copy(acc_sh.at[pl.ds(s * SR, SR)],
                    out_hbm.at[pl.ds(c * NH + s * SR, SR)])


@functools.cache
def _edge_kernel():
    return pl.kernel(
        _edge_body,
        out_type=jax.ShapeDtypeStruct((2 * NH, D), jnp.float32),
        mesh=_sc_mesh(),
        scratch_types=[
            pltpu.VMEM_SHARED((NHT, D), jnp.float32),
            pltpu.VMEM((CH,), jnp.int32),
            pltpu.VMEM((CH,), jnp.int32),
            pltpu.VMEM((CH, D), jnp.float32),
            pltpu.VMEM((SR, D), jnp.float32),
            pltpu.SemaphoreType.DMA,
        ],
    )


def _tc_head_body(deg_ref, x_ref, w_ref, dinv_ref, z_ref):
    deg = deg_ref[0:N, 0:1] + 1.0
    dinv = lax.rsqrt(deg)
    hw = jnp.dot(x_ref[...], w_ref[...], preferred_element_type=jnp.float32)
    dinv_ref[...] = dinv
    z_ref[...] = hw * dinv


def _tc_head(deg, x, w):
    return pl.pallas_call(
        _tc_head_body,
        out_shape=(jax.ShapeDtypeStruct((N, 1), jnp.float32),
                   jax.ShapeDtypeStruct((N, D), jnp.float32)),
    )(deg, x, w)


def _tc_mid_body(s_ref, z_ref, dinv_ref, b_ref, w_ref, zn_ref):
    dinv = dinv_ref[...]
    h = jnp.tanh((s_ref[0:N, :] + z_ref[...]) * dinv + b_ref[...])
    hw = jnp.dot(h, w_ref[...], preferred_element_type=jnp.float32)
    zn_ref[...] = hw * dinv


def _tc_mid(s, z, dinv, b, w):
    return pl.pallas_call(
        _tc_mid_body,
        out_shape=jax.ShapeDtypeStruct((N, D), jnp.float32),
    )(s, z, dinv, b, w)


def _tc_tail_body(s_ref, z_ref, dinv_ref, b_ref, batch_ref,
                  fc1w_ref, fc1b_ref, fc2w_ref, fc2b_ref, fc3w_ref, fc3b_ref,
                  out_ref):
    h = jnp.tanh((s_ref[0:N, :] + z_ref[...]) * dinv_ref[...] + b_ref[...])
    t = jnp.tanh(jnp.dot(h, fc1w_ref[...],
                         preferred_element_type=jnp.float32) + fc1b_ref[...])
    t = jnp.tanh(jnp.dot(t, fc2w_ref[...],
                         preferred_element_type=jnp.float32) + fc2b_ref[...])
    y = jnp.dot(t, fc3w_ref[...],
                preferred_element_type=jnp.float32) + fc3b_ref[...]
    b_ids = batch_ref[...]
    gids = lax.broadcasted_iota(jnp.int32, (G, N), 0)
    mask = (b_ids[None, :] == gids).astype(jnp.float32)
    ssum = jnp.dot(mask, y, preferred_element_type=jnp.float32)
    cnt = jnp.sum(mask, axis=1, keepdims=True)
    out_ref[...] = jax.nn.sigmoid(ssum / jnp.maximum(cnt, 1.0))


def _tc_tail(s, z, dinv, b, batch, fc1w, fc1b, fc2w, fc2b, fc3w, fc3b):
    return pl.pallas_call(
        _tc_tail_body,
        out_shape=jax.ShapeDtypeStruct((G, 1), jnp.float32),
    )(s, z, dinv, b, batch, fc1w, fc1b, fc2w, fc2b, fc3w, fc3b)


def kernel(x, edge_index, batch, W_in, b_in, W_h0, b_h0, W_h1, b_h1,
           fc1_W, fc1_b, fc2_W, fc2_b, fc3_W, fc3_b):
    src = edge_index[0]
    dst = edge_index[1]

    deg16 = _deg_kernel()(dst)
    dinv, z0 = _tc_head(deg16[:, 0:1], x, W_in)

    s0 = _edge_kernel()(z0, src, dst)
    z1 = _tc_mid(s0, z0, dinv, b_in.reshape(1, D), W_h0)

    s1 = _edge_kernel()(z1, src, dst)
    z2 = _tc_mid(s1, z1, dinv, b_h0.reshape(1, D), W_h1)

    s2 = _edge_kernel()(z2, src, dst)
    return _tc_tail(s2, z2, dinv, b_h1.reshape(1, D), batch,
                    fc1_W, fc1_b.reshape(1, D), fc2_W, fc2_b.reshape(1, 32),
                    fc3_W, fc3_b.reshape(1, 1))


# speedup vs baseline: 11.9817x; 2.0083x over previous
import functools
import jax
import jax.numpy as jnp
from jax import lax
from jax.experimental import pallas as pl
from jax.experimental.pallas import tpu as pltpu
from jax.experimental.pallas import tpu_sc as plsc

N = 10000
E = 320000
D = 128
G = 64

NC = 2
NS = 16
NH = 5120
NHT = NH + 8
SR = NH // NS
ET = E // NS
CH = 80
NCH = ET // CH


def _remap_dst(didx_v, c):
    base = c * NH
    for k in range(CH // 16):
        v = didx_v[pl.ds(k * 16, 16)] - base
        ok = (v >= 0) & (v < NH)
        didx_v[pl.ds(k * 16, 16)] = jnp.where(ok, v, NH)


def _deg_body(dst_hbm, out_hbm, acc_sh, didx_v, ones_v, zb_v):
    c = lax.axis_index("c")
    s = lax.axis_index("s")
    base = s * ET

    one16 = jnp.full((16,), 1.0, jnp.float32)
    zero16 = jnp.zeros((16,), jnp.float32)

    def fill(i, _):
        ones_v[i] = one16
        return 0
    lax.fori_loop(0, CH, fill, 0)

    def zfill(i, _):
        zb_v[i] = zero16
        return 0
    lax.fori_loop(0, SR, zfill, 0)
    pltpu.sync_copy(zb_v, acc_sh.at[pl.ds(s * SR, SR)])

    @pl.when(s == NS - 1)
    def _():
        pltpu.sync_copy(zb_v.at[pl.ds(0, 8)], acc_sh.at[pl.ds(NH, 8)])

    plsc.subcore_barrier()

    def chunk(j, _):
        pltpu.sync_copy(dst_hbm.at[pl.ds(base + j * CH, CH)], didx_v)
        _remap_dst(didx_v, c)
        pltpu.sync_copy(ones_v, acc_sh.at[didx_v], add=True)
        return 0
    lax.fori_loop(0, NCH, chunk, 0)
    plsc.subcore_barrier()

    pltpu.sync_copy(acc_sh.at[pl.ds(s * SR, SR)],
                    out_hbm.at[pl.ds(c * NH + s * SR, SR)])


@functools.cache
def _sc_mesh():
    return plsc.VectorSubcoreMesh(core_axis_name="c", subcore_axis_name="s",
                                  num_cores=NC, num_subcores=NS)


@functools.cache
def _deg_kernel():
    return pl.kernel(
        _deg_body,
        out_type=jax.ShapeDtypeStruct((2 * NH, 16), jnp.float32),
        mesh=_sc_mesh(),
        scratch_types=[
            pltpu.VMEM_SHARED((NHT, 16), jnp.float32),
            pltpu.VMEM((CH,), jnp.int32),
            pltpu.VMEM((CH, 16), jnp.float32),
            pltpu.VMEM((SR, 16), jnp.float32),
        ],
    )


def _edge_body(z_hbm, src_hbm, dst_hbm, out_hbm,
               acc_sh, sidx_v, didx_v, rows_a, rows_b, sem_a, sem_b):
    c = lax.axis_index("c")
    s = lax.axis_index("s")

    zero16 = jnp.zeros((16,), jnp.float32)

    def zfill(i, _):
        for j in range(8):
            rows_a[i, pl.ds(j * 16, 16)] = zero16
        return 0
    lax.fori_loop(0, CH, zfill, 0)
    for r in range(SR // CH):
        pltpu.sync_copy(rows_a, acc_sh.at[pl.ds(s * SR + r * CH, CH)])

    @pl.when(s == NS - 1)
    def _():
        pltpu.sync_copy(rows_a.at[pl.ds(0, 8)], acc_sh.at[pl.ds(NH, 8)])

    pltpu.sync_copy(src_hbm.at[s], sidx_v)
    pltpu.sync_copy(dst_hbm.at[s], didx_v)
    base = c * NH

    def remap(r, _):
        for k in range(CH // 16):
            v = didx_v[r, pl.ds(k * 16, 16)] - base
            ok = (v >= 0) & (v < NH)
            didx_v[r, pl.ds(k * 16, 16)] = jnp.where(ok, v, NH)
        return 0
    lax.fori_loop(0, NCH, remap, 0)
    plsc.subcore_barrier()

    pltpu.async_copy(z_hbm.at[sidx_v.at[0]], rows_a, sem_a)

    def chunk2(i, _):
        j = 2 * i
        pltpu.async_copy(z_hbm.at[sidx_v.at[j + 1]], rows_b, sem_b)
        pltpu.make_async_copy(z_hbm.at[sidx_v.at[j]], rows_a, sem_a).wait()
        pltpu.sync_copy(rows_a, acc_sh.at[didx_v.at[j]], add=True)

        @pl.when(j + 2 < NCH)
        def _():
            pltpu.async_copy(z_hbm.at[sidx_v.at[j + 2]], rows_a, sem_a)

        pltpu.make_async_copy(z_hbm.at[sidx_v.at[j + 1]], rows_b, sem_b).wait()
        pltpu.sync_copy(rows_b, acc_sh.at[didx_v.at[j + 1]], add=True)
        return 0
    lax.fori_loop(0, NCH // 2, chunk2, 0)
    plsc.subcore_barrier()

    pltpu.sync_copy(acc_sh.at[pl.ds(s * SR, SR)],
                    out_hbm.at[pl.ds(c * NH + s * SR, SR)])


@functools.cache
def _edge_kernel():
    return pl.kernel(
        _edge_body,
        out_type=jax.ShapeDtypeStruct((2 * NH, D), jnp.float32),
        mesh=_sc_mesh(),
        scratch_types=[
            pltpu.VMEM_SHARED((NHT, D), jnp.float32),
            pltpu.VMEM((NCH, CH), jnp.int32),
            pltpu.VMEM((NCH, CH), jnp.int32),
            pltpu.VMEM((CH, D), jnp.float32),
            pltpu.VMEM((CH, D), jnp.float32),
            pltpu.SemaphoreType.DMA,
            pltpu.SemaphoreType.DMA,
        ],
    )


def _tc_head_body(deg_ref, x_ref, w_ref, dinv_ref, z_ref):
    deg = deg_ref[0:N, 0:1] + 1.0
    dinv = lax.rsqrt(deg)
    hw = jnp.dot(x_ref[...], w_ref[...], preferred_element_type=jnp.float32)
    dinv_ref[...] = dinv
    z_ref[...] = hw * dinv


def _tc_head(deg, x, w):
    return pl.pallas_call(
        _tc_head_body,
        out_shape=(jax.ShapeDtypeStruct((N, 1), jnp.float32),
                   jax.ShapeDtypeStruct((N, D), jnp.float32)),
    )(deg, x, w)


def _tc_mid_body(s_ref, z_ref, dinv_ref, b_ref, w_ref, zn_ref):
    dinv = dinv_ref[...]
    h = jnp.tanh((s_ref[0:N, :] + z_ref[...]) * dinv + b_ref[...])
    hw = jnp.dot(h, w_ref[...], preferred_element_type=jnp.float32)
    zn_ref[...] = hw * dinv


def _tc_mid(s, z, dinv, b, w):
    return pl.pallas_call(
        _tc_mid_body,
        out_shape=jax.ShapeDtypeStruct((N, D), jnp.float32),
    )(s, z, dinv, b, w)


def _tc_tail_body(s_ref, z_ref, dinv_ref, b_ref, batch_ref,
                  fc1w_ref, fc1b_ref, fc2w_ref, fc2b_ref, fc3w_ref, fc3b_ref,
                  out_ref):
    h = jnp.tanh((s_ref[0:N, :] + z_ref[...]) * dinv_ref[...] + b_ref[...])
    t = jnp.tanh(jnp.dot(h, fc1w_ref[...],
                         preferred_element_type=jnp.float32) + fc1b_ref[...])
    t = jnp.tanh(jnp.dot(t, fc2w_ref[...],
                         preferred_element_type=jnp.float32) + fc2b_ref[...])
    y = jnp.dot(t, fc3w_ref[...],
                preferred_element_type=jnp.float32) + fc3b_ref[...]
    b_ids = batch_ref[...]
    gids = lax.broadcasted_iota(jnp.int32, (G, N), 0)
    mask = (b_ids[None, :] == gids).astype(jnp.float32)
    ssum = jnp.dot(mask, y, preferred_element_type=jnp.float32)
    cnt = jnp.sum(mask, axis=1, keepdims=True)
    out_ref[...] = jax.nn.sigmoid(ssum / jnp.maximum(cnt, 1.0))


def _tc_tail(s, z, dinv, b, batch, fc1w, fc1b, fc2w, fc2b, fc3w, fc3b):
    return pl.pallas_call(
        _tc_tail_body,
        out_shape=jax.ShapeDtypeStruct((G, 1), jnp.float32),
    )(s, z, dinv, b, batch, fc1w, fc1b, fc2w, fc2b, fc3w, fc3b)


def kernel(x, edge_index, batch, W_in, b_in, W_h0, b_h0, W_h1, b_h1,
           fc1_W, fc1_b, fc2_W, fc2_b, fc3_W, fc3_b):
    src = edge_index[0]
    dst = edge_index[1]
    src3 = src.reshape(NS, NCH, CH)
    dst3 = dst.reshape(NS, NCH, CH)

    deg16 = _deg_kernel()(dst)
    dinv, z0 = _tc_head(deg16[:, 0:1], x, W_in)

    s0 = _edge_kernel()(z0, src3, dst3)
    z1 = _tc_mid(s0, z0, dinv, b_in.reshape(1, D), W_h0)

    s1 = _edge_kernel()(z1, src3, dst3)
    z2 = _tc_mid(s1, z1, dinv, b_h0.reshape(1, D), W_h1)

    s2 = _edge_kernel()(z2, src3, dst3)
    return _tc_tail(s2, z2, dinv, b_h1.reshape(1, D), batch,
                    fc1_W, fc1_b.reshape(1, D), fc2_W, fc2_b.reshape(1, 32),
                    fc3_W, fc3_b.reshape(1, 1))
